# baseline (speedup 1.0000x reference)
import functools

import jax
import jax.numpy as jnp
from jax import lax
from jax.experimental import pallas as pl
from jax.experimental.pallas import tpu as pltpu
from jax.experimental.pallas import tpu_sc as plsc

VOCAB = 100000
D = 128
B = 1024
MAX_NORM = 1.0
VT = 2048


def _gather_sc(table, idx):
    info = plsc.get_sparse_core_info()
    nc, ns = info.num_cores, info.num_subcores
    nw = nc * ns
    b_per_w = B // nw
    mesh = plsc.VectorSubcoreMesh(core_axis_name="c", subcore_axis_name="s")

    @functools.partial(
        pl.kernel,
        mesh=mesh,
        out_type=jax.ShapeDtypeStruct((B, D), jnp.float32),
        scratch_types=[
            pltpu.VMEM((b_per_w,), jnp.int32),
            pltpu.VMEM((b_per_w, D), jnp.float32),
            pltpu.SemaphoreType.DMA,
        ],
    )
    def k(table_hbm, idx_hbm, out_hbm, idx_v, rows_v, sem):
        wid = lax.axis_index("s") * nc + lax.axis_index("c")
        base = wid * b_per_w
        pltpu.sync_copy(idx_hbm.at[pl.ds(base, b_per_w)], idx_v)
        pltpu.async_copy(table_hbm.at[idx_v], rows_v, sem).wait()
        pltpu.sync_copy(rows_v, out_hbm.at[pl.ds(base, b_per_w)])

    return k(table, idx)


def _project_tc(e, W, b2):
    nv = pl.cdiv(VOCAB, VT)

    def body(e_ref, w_ref, b_ref, o_ref, x_ref):
        @pl.when(pl.program_id(0) == 0)
        def _():
            ev = e_ref[...]
            ss = jnp.sum(ev * ev, axis=1, keepdims=True)
            nrm = jnp.sqrt(ss)
            scale = jnp.where(nrm > MAX_NORM, MAX_NORM / (nrm + 1e-7), 1.0)
            x_ref[...] = ev * scale

        o_ref[...] = lax.dot_general(
            x_ref[...], w_ref[...],
            dimension_numbers=(((1,), (1,)), ((), ())),
            preferred_element_type=jnp.float32,
        ) + b_ref[...]

    return pl.pallas_call(
        body,
        grid=(nv,),
        in_specs=[
            pl.BlockSpec((B, D), lambda i: (0, 0)),
            pl.BlockSpec((VT, D), lambda i: (i, 0)),
            pl.BlockSpec((1, VT), lambda i: (0, i)),
        ],
        out_specs=pl.BlockSpec((B, VT), lambda i: (0, i)),
        out_shape=jax.ShapeDtypeStruct((B, VOCAB), jnp.float32),
        scratch_shapes=[pltpu.VMEM((B, D), jnp.float32)],
    )(e, W, b2)


def kernel(inputs_, emb_table, W, b):
    idx = inputs_.astype(jnp.int32)
    e = _gather_sc(emb_table, idx)
    return _project_tc(e, W, b.reshape(1, VOCAB))


# speedup vs baseline: 2.3452x; 2.3452x over previous
import functools

import jax
import jax.numpy as jnp
from jax import lax
from jax.experimental import pallas as pl
from jax.experimental.pallas import tpu as pltpu
from jax.experimental.pallas import tpu_sc as plsc

VOCAB = 100000
D = 128
B = 1024
MAX_NORM = 1.0
VT = 2048


def _gather_sc(table, idx):
    info = plsc.get_sparse_core_info()
    nc, ns = info.num_cores, info.num_subcores
    nw = nc * ns
    b_per_w = B // nw
    mesh = plsc.VectorSubcoreMesh(core_axis_name="c", subcore_axis_name="s")

    @functools.partial(
        pl.kernel,
        mesh=mesh,
        out_type=jax.ShapeDtypeStruct((B, D), jnp.float32),
        scratch_types=[
            pltpu.VMEM((b_per_w,), jnp.int32),
            pltpu.VMEM((b_per_w, D), jnp.float32),
            pltpu.SemaphoreType.DMA,
        ],
    )
    def k(table_hbm, idx_hbm, out_hbm, idx_v, rows_v, sem):
        wid = lax.axis_index("s") * nc + lax.axis_index("c")
        base = wid * b_per_w
        pltpu.sync_copy(idx_hbm.at[pl.ds(base, b_per_w)], idx_v)
        pltpu.async_copy(table_hbm.at[idx_v], rows_v, sem).wait()
        pltpu.sync_copy(rows_v, out_hbm.at[pl.ds(base, b_per_w)])

    return k(table, idx)


def _project_tc(e, W, b2):
    nv = pl.cdiv(VOCAB, VT)

    def body(e_ref, w_ref, b_ref, o_ref, x_ref):
        @pl.when(pl.program_id(0) == 0)
        def _():
            ev = e_ref[...]
            ss = jnp.sum(ev * ev, axis=1, keepdims=True)
            nrm = jnp.sqrt(ss)
            scale = jnp.where(nrm > MAX_NORM, MAX_NORM / (nrm + 1e-7), 1.0)
            x_ref[...] = ev * scale

        o_ref[...] = lax.dot_general(
            w_ref[...], x_ref[...],
            dimension_numbers=(((1,), (1,)), ((), ())),
            preferred_element_type=jnp.float32,
        ) + b_ref[...]

    return pl.pallas_call(
        body,
        grid=(nv,),
        in_specs=[
            pl.BlockSpec((B, D), lambda i: (0, 0)),
            pl.BlockSpec((VT, D), lambda i: (i, 0)),
            pl.BlockSpec((VT, 1), lambda i: (i, 0)),
        ],
        out_specs=pl.BlockSpec((VT, B), lambda i: (i, 0)),
        out_shape=jax.ShapeDtypeStruct((VOCAB, B), jnp.float32),
        scratch_shapes=[pltpu.VMEM((B, D), jnp.float32)],
    )(e, W, b2)


def kernel(inputs_, emb_table, W, b):
    idx = inputs_.astype(jnp.int32)
    e = _gather_sc(emb_table, idx)
    out_t = _project_tc(e, W, b.reshape(VOCAB, 1))
    return out_t.T


# speedup vs baseline: 3.1258x; 1.3329x over previous
import functools

import jax
import jax.numpy as jnp
from jax import lax
from jax.experimental import pallas as pl
from jax.experimental.pallas import tpu as pltpu
from jax.experimental.pallas import tpu_sc as plsc

VOCAB = 100000
D = 128
B = 1024
MAX_NORM = 1.0
VT = 2048


def _gather_sc(table, idx):
    info = plsc.get_sparse_core_info()
    nc, ns = info.num_cores, info.num_subcores
    nw = nc * ns
    b_per_w = B // nw
    mesh = plsc.VectorSubcoreMesh(core_axis_name="c", subcore_axis_name="s")

    @functools.partial(
        pl.kernel,
        mesh=mesh,
        out_type=jax.ShapeDtypeStruct((B, D), jnp.float32),
        scratch_types=[
            pltpu.VMEM((b_per_w,), jnp.int32),
            pltpu.VMEM((b_per_w, D), jnp.float32),
            pltpu.SemaphoreType.DMA,
        ],
    )
    def k(table_hbm, idx_hbm, out_hbm, idx_v, rows_v, sem):
        wid = lax.axis_index("s") * nc + lax.axis_index("c")
        base = wid * b_per_w
        pltpu.sync_copy(idx_hbm.at[pl.ds(base, b_per_w)], idx_v)
        pltpu.async_copy(table_hbm.at[idx_v], rows_v, sem).wait()
        pltpu.sync_copy(rows_v, out_hbm.at[pl.ds(base, b_per_w)])

    return k(table, idx)


def _project_tc(e, W, b2):
    nv = pl.cdiv(VOCAB, VT)

    def body(e_ref, w_ref, b_ref, o_ref, x_ref):
        @pl.when(pl.program_id(0) == 0)
        def _():
            ev = e_ref[...]
            ss = jnp.sum(ev * ev, axis=1, keepdims=True)
            nrm = jnp.sqrt(ss)
            scale = jnp.where(nrm > MAX_NORM, MAX_NORM / (nrm + 1e-7), 1.0)
            x_ref[...] = ev * scale

        o_ref[...] = lax.dot_general(
            w_ref[...], x_ref[...],
            dimension_numbers=(((1,), (1,)), ((), ())),
            preferred_element_type=jnp.float32,
        ) + b_ref[...][:, None]

    return pl.pallas_call(
        body,
        grid=(nv,),
        in_specs=[
            pl.BlockSpec((B, D), lambda i: (0, 0)),
            pl.BlockSpec((VT, D), lambda i: (i, 0)),
            pl.BlockSpec((VT,), lambda i: (i,)),
        ],
        out_specs=pl.BlockSpec((VT, B), lambda i: (i, 0)),
        out_shape=jax.ShapeDtypeStruct((VOCAB, B), jnp.float32),
        scratch_shapes=[pltpu.VMEM((B, D), jnp.float32)],
    )(e, W, b2)


def kernel(inputs_, emb_table, W, b):
    idx = inputs_.astype(jnp.int32)
    e = _gather_sc(emb_table, idx)
    out_t = _project_tc(e, W, b)
    return out_t.T


# speedup vs baseline: 3.2031x; 1.0247x over previous
import functools

import jax
import jax.numpy as jnp
from jax import lax
from jax.experimental import pallas as pl
from jax.experimental.pallas import tpu as pltpu
from jax.experimental.pallas import tpu_sc as plsc

VOCAB = 100000
D = 128
B = 1024
MAX_NORM = 1.0
VT = 4096


def _gather_sc(table, idx):
    info = plsc.get_sparse_core_info()
    nc, ns = info.num_cores, info.num_subcores
    nw = nc * ns
    b_per_w = B // nw
    mesh = plsc.VectorSubcoreMesh(core_axis_name="c", subcore_axis_name="s")

    @functools.partial(
        pl.kernel,
        mesh=mesh,
        out_type=jax.ShapeDtypeStruct((B, D), jnp.float32),
        scratch_types=[
            pltpu.VMEM((b_per_w,), jnp.int32),
            pltpu.VMEM((b_per_w, D), jnp.float32),
            pltpu.SemaphoreType.DMA,
        ],
    )
    def k(table_hbm, idx_hbm, out_hbm, idx_v, rows_v, sem):
        wid = lax.axis_index("s") * nc + lax.axis_index("c")
        base = wid * b_per_w
        pltpu.sync_copy(idx_hbm.at[pl.ds(base, b_per_w)], idx_v)
        pltpu.async_copy(table_hbm.at[idx_v], rows_v, sem).wait()
        pltpu.sync_copy(rows_v, out_hbm.at[pl.ds(base, b_per_w)])

    return k(table, idx)


def _project_tc(e, W, b2):
    nv = pl.cdiv(VOCAB, VT)

    def body(e_ref, w_ref, b_ref, o_ref, x_ref):
        @pl.when(pl.program_id(0) == 0)
        def _():
            ev = e_ref[...]
            ss = jnp.sum(ev * ev, axis=1, keepdims=True)
            nrm = jnp.sqrt(ss)
            scale = jnp.where(nrm > MAX_NORM, MAX_NORM / (nrm + 1e-7), 1.0)
            x_ref[...] = ev * scale

        o_ref[...] = lax.dot_general(
            w_ref[...], x_ref[...],
            dimension_numbers=(((1,), (1,)), ((), ())),
            preferred_element_type=jnp.float32,
        ) + b_ref[...][:, None]

    return pl.pallas_call(
        body,
        grid=(nv,),
        in_specs=[
            pl.BlockSpec((B, D), lambda i: (0, 0)),
            pl.BlockSpec((VT, D), lambda i: (i, 0)),
            pl.BlockSpec((VT,), lambda i: (i,)),
        ],
        out_specs=pl.BlockSpec((VT, B), lambda i: (i, 0)),
        out_shape=jax.ShapeDtypeStruct((VOCAB, B), jnp.float32),
        scratch_shapes=[pltpu.VMEM((B, D), jnp.float32)],
    )(e, W, b2)


def kernel(inputs_, emb_table, W, b):
    idx = inputs_.astype(jnp.int32)
    e = _gather_sc(emb_table, idx)
    out_t = _project_tc(e, W, b)
    return out_t.T


# speedup vs baseline: 3.2122x; 1.0028x over previous
import functools

import jax
import jax.numpy as jnp
from jax import lax
from jax.experimental import pallas as pl
from jax.experimental.pallas import tpu as pltpu
from jax.experimental.pallas import tpu_sc as plsc

VOCAB = 100000
D = 128
B = 1024
MAX_NORM = 1.0
VT = 5120


def _gather_sc(table, idx):
    info = plsc.get_sparse_core_info()
    nc, ns = info.num_cores, info.num_subcores
    nw = nc * ns
    b_per_w = B // nw
    mesh = plsc.VectorSubcoreMesh(core_axis_name="c", subcore_axis_name="s")

    @functools.partial(
        pl.kernel,
        mesh=mesh,
        out_type=jax.ShapeDtypeStruct((B, D), jnp.float32),
        scratch_types=[
            pltpu.VMEM((b_per_w,), jnp.int32),
            pltpu.VMEM((b_per_w, D), jnp.float32),
            pltpu.SemaphoreType.DMA,
        ],
    )
    def k(table_hbm, idx_hbm, out_hbm, idx_v, rows_v, sem):
        wid = lax.axis_index("s") * nc + lax.axis_index("c")
        base = wid * b_per_w
        pltpu.sync_copy(idx_hbm.at[pl.ds(base, b_per_w)], idx_v)
        pltpu.async_copy(table_hbm.at[idx_v], rows_v, sem).wait()
        pltpu.sync_copy(rows_v, out_hbm.at[pl.ds(base, b_per_w)])

    return k(table, idx)


def _project_tc(e, W, b2):
    nv = pl.cdiv(VOCAB, VT)

    def body(e_ref, w_ref, b_ref, o_ref, x_ref):
        @pl.when(pl.program_id(0) == 0)
        def _():
            ev = e_ref[...]
            ss = jnp.sum(ev * ev, axis=1, keepdims=True)
            nrm = jnp.sqrt(ss)
            scale = jnp.where(nrm > MAX_NORM, MAX_NORM / (nrm + 1e-7), 1.0)
            x_ref[...] = ev * scale

        o_ref[...] = lax.dot_general(
            w_ref[...], x_ref[...],
            dimension_numbers=(((1,), (1,)), ((), ())),
            preferred_element_type=jnp.float32,
        ) + b_ref[...][:, None]

    return pl.pallas_call(
        body,
        grid=(nv,),
        in_specs=[
            pl.BlockSpec((B, D), lambda i: (0, 0)),
            pl.BlockSpec((VT, D), lambda i: (i, 0)),
            pl.BlockSpec((VT,), lambda i: (i,)),
        ],
        out_specs=pl.BlockSpec((VT, B), lambda i: (i, 0)),
        out_shape=jax.ShapeDtypeStruct((VOCAB, B), jnp.float32),
        scratch_shapes=[pltpu.VMEM((B, D), jnp.float32)],
    )(e, W, b2)


def kernel(inputs_, emb_table, W, b):
    idx = inputs_.astype(jnp.int32)
    e = _gather_sc(emb_table, idx)
    out_t = _project_tc(e, W, b)
    return out_t.T
